# baseline (speedup 1.0000x reference)
import functools

import jax
import jax.numpy as jnp
from jax import lax
from jax.experimental import pallas as pl
from jax.experimental.pallas import tpu as pltpu
from jax.experimental.pallas import tpu_sc as plsc

N = 10000
NP = 10240
E = 320000
NC = 2
NS = 16
CH = 80
NCH_E = E // (NC * NS * CH)
NCH_F = E // (NS * CH)
RPT = NP // NS
SCK = 25
D = 128
DEGW = 16
BM = 1024
NBLK = NP // BM


def _sc_mesh():
    return plsc.VectorSubcoreMesh(core_axis_name="c", subcore_axis_name="s")


@functools.cache
def _deg_call():
    @functools.partial(
        pl.kernel,
        out_type=jax.ShapeDtypeStruct((NC, NP, DEGW), jnp.float32),
        mesh=_sc_mesh(),
        scratch_types=[
            pltpu.VMEM((NCH_E, CH), jnp.int32),
            pltpu.VMEM((CH, DEGW), jnp.float32),
            pltpu.VMEM_SHARED((NP, DEGW), jnp.float32),
            pltpu.SemaphoreType.DMA,
        ],
    )
    def deg_kernel(dstix, zeros, deg_out, didx_v, ones_v, acc, ssem):
        c = lax.axis_index("c")
        s = lax.axis_index("s")
        w = c * NS + s
        r0 = s * RPT

        @pl.loop(0, CH)
        def _(i):
            ones_v[i, :] = jnp.full((DEGW,), 1.0, jnp.float32)

        pltpu.sync_copy(zeros.at[pl.ds(r0, RPT)], acc.at[pl.ds(r0, RPT)])
        pltpu.sync_copy(dstix.at[w], didx_v)
        plsc.subcore_barrier()

        @pl.loop(0, NCH_E)
        def _(j):
            pltpu.async_copy(ones_v, acc.at[didx_v.at[j]], ssem, add=True)

        @pl.loop(0, NCH_E)
        def _(j):
            pltpu.make_async_copy(ones_v, acc.at[didx_v.at[0]], ssem).wait()

        plsc.subcore_barrier()
        pltpu.sync_copy(acc.at[pl.ds(r0, RPT)], deg_out.at[c, pl.ds(r0, RPT)])

    return deg_kernel


@functools.cache
def _prop_call(split_edges):
    nch = NCH_E if split_edges else NCH_F
    outer = nch // SCK

    @functools.partial(
        pl.kernel,
        out_type=jax.ShapeDtypeStruct((NC * NP, D), jnp.float32),
        mesh=_sc_mesh(),
        scratch_types=[
            pltpu.VMEM((SCK, CH), jnp.int32),
            pltpu.VMEM((SCK, CH), jnp.int32),
            pltpu.VMEM((4, CH, D), jnp.float32),
            pltpu.VMEM_SHARED((NP, D), jnp.float32),
            pltpu.SemaphoreType.DMA((4,)),
            pltpu.SemaphoreType.DMA((4,)),
        ],
    )
    def prop_kernel(srcix, dstix, table, zeros, out,
                    sidx_v, didx_v, msgs, acc, gsem, ssem):
        c = lax.axis_index("c")
        s = lax.axis_index("s")
        r0 = s * RPT
        cN = c * NP

        pltpu.sync_copy(zeros.at[pl.ds(r0, RPT)], acc.at[pl.ds(r0, RPT)])
        plsc.subcore_barrier()

        @pl.loop(0, outer)
        def _(o):
            if split_edges:
                w = c * NS + s
                pltpu.sync_copy(srcix.at[w, o], sidx_v)
                pltpu.sync_copy(dstix.at[w, o], didx_v)
            else:
                pltpu.sync_copy(srcix.at[s, o], sidx_v)
                pltpu.sync_copy(dstix.at[s, o], didx_v)

                @pl.loop(0, SCK)
                def _(i):
                    @pl.loop(0, CH // 16)
                    def _(k):
                        sidx_v[i, pl.ds(k * 16, 16)] = (
                            sidx_v[i, pl.ds(k * 16, 16)] + cN)

            @pl.loop(0, SCK + 1)
            def _(j):
                b = lax.rem(j, 4)

                @pl.when(j < SCK)
                def _():
                    @pl.when(j >= 4)
                    def _():
                        pltpu.make_async_copy(
                            msgs.at[b], acc.at[didx_v.at[0]],
                            ssem.at[b]).wait()

                    pltpu.async_copy(table.at[sidx_v.at[j]],
                                     msgs.at[b], gsem.at[b])

                @pl.when(j >= 1)
                def _():
                    pb = lax.rem(j + 3, 4)
                    pltpu.make_async_copy(table.at[sidx_v.at[0]],
                                          msgs.at[pb], gsem.at[pb]).wait()
                    pltpu.async_copy(msgs.at[pb], acc.at[didx_v.at[j - 1]],
                                     ssem.at[pb], add=True)

            for bb in range(4):
                pltpu.make_async_copy(msgs.at[bb], acc.at[didx_v.at[0]],
                                      ssem.at[bb]).wait()

        plsc.subcore_barrier()
        pltpu.sync_copy(acc.at[pl.ds(r0, RPT)], out.at[pl.ds(cN + r0, RPT)])

    return prop_kernel


def _scale_body(x_ref, da_ref, db_ref, y_ref, dinv_ref):
    deg = da_ref[0, :, :1] + db_ref[0, :, :1] + 1.0
    dinv = lax.rsqrt(deg)
    y_ref[...] = x_ref[...] * dinv
    dinv_ref[...] = jnp.broadcast_to(dinv, (BM, D))


@functools.cache
def _scale_call():
    return pl.pallas_call(
        _scale_body,
        grid=(NBLK,),
        in_specs=[
            pl.BlockSpec((BM, D), lambda i: (i, 0)),
            pl.BlockSpec((1, BM, DEGW), lambda i: (0, i, 0)),
            pl.BlockSpec((1, BM, DEGW), lambda i: (1, i, 0)),
        ],
        out_specs=[
            pl.BlockSpec((BM, D), lambda i: (i, 0)),
            pl.BlockSpec((BM, D), lambda i: (i, 0)),
        ],
        out_shape=[
            jax.ShapeDtypeStruct((NP, D), jnp.float32),
            jax.ShapeDtypeStruct((NP, D), jnp.float32),
        ],
    )


def _layer1_body(a0_ref, a1_ref, y_ref, dinv_ref, w_ref, b_ref, ycat_ref):
    dinv = dinv_ref[...]
    z = (a0_ref[...] + a1_ref[...] + y_ref[...]) * dinv
    h = jnp.dot(z, w_ref[...], preferred_element_type=jnp.float32) + b_ref[...]
    h = jnp.maximum(h, 0.0)
    ycat_ref[0] = h[:, :D] * dinv
    ycat_ref[1] = h[:, D:] * dinv


@functools.cache
def _layer1_call():
    return pl.pallas_call(
        _layer1_body,
        grid=(NBLK,),
        in_specs=[
            pl.BlockSpec((BM, D), lambda i: (i, 0)),
            pl.BlockSpec((BM, D), lambda i: (i + NBLK, 0)),
            pl.BlockSpec((BM, D), lambda i: (i, 0)),
            pl.BlockSpec((BM, D), lambda i: (i, 0)),
            pl.BlockSpec((D, 256), lambda i: (0, 0)),
            pl.BlockSpec((1, 256), lambda i: (0, 0)),
        ],
        out_specs=pl.BlockSpec((2, BM, D), lambda i: (0, i, 0)),
        out_shape=jax.ShapeDtypeStruct((2, NP, D), jnp.float32),
    )


def _layer2_body(a_lo, a_hi, y_lo, y_hi, dinv_ref, w_ref, b_ref, ycat_ref):
    dinv = dinv_ref[...]
    z = jnp.concatenate(
        [(a_lo[...] + y_lo[...]) * dinv, (a_hi[...] + y_hi[...]) * dinv],
        axis=1)
    h = jnp.dot(z, w_ref[...], preferred_element_type=jnp.float32) + b_ref[...]
    h = jnp.maximum(h, 0.0)
    ycat_ref[0] = h[:, :D] * dinv
    ycat_ref[1] = h[:, D:] * dinv


def _final_body(a_lo, a_hi, y_lo, y_hi, dinv_ref, w_ref, b_ref, out_ref):
    dinv = dinv_ref[...]
    z = jnp.concatenate(
        [(a_lo[...] + y_lo[...]) * dinv, (a_hi[...] + y_hi[...]) * dinv],
        axis=1)
    out_ref[...] = (
        jnp.dot(z, w_ref[...], preferred_element_type=jnp.float32) + b_ref[...])


def _wide_in_specs():
    return [
        pl.BlockSpec((BM, D), lambda i: (i, 0)),
        pl.BlockSpec((BM, D), lambda i: (i + NBLK, 0)),
        pl.BlockSpec((BM, D), lambda i: (i, 0)),
        pl.BlockSpec((BM, D), lambda i: (i + NBLK, 0)),
        pl.BlockSpec((BM, D), lambda i: (i, 0)),
        pl.BlockSpec((256, 256), lambda i: (0, 0)),
        pl.BlockSpec((1, 256), lambda i: (0, 0)),
    ]


@functools.cache
def _layer2_call():
    return pl.pallas_call(
        _layer2_body,
        grid=(NBLK,),
        in_specs=_wide_in_specs(),
        out_specs=pl.BlockSpec((2, BM, D), lambda i: (0, i, 0)),
        out_shape=jax.ShapeDtypeStruct((2, NP, D), jnp.float32),
    )


@functools.cache
def _final_call():
    return pl.pallas_call(
        _final_body,
        grid=(NBLK,),
        in_specs=_wide_in_specs(),
        out_specs=pl.BlockSpec((BM, 256), lambda i: (i, 0)),
        out_shape=jax.ShapeDtypeStruct((N, 256), jnp.float32),
    )


def kernel(x, edge_index, W0, b0, W1, b1):
    src = edge_index[0].astype(jnp.int32)
    dst = edge_index[1].astype(jnp.int32)
    dstix_d = dst.reshape(NC * NS, NCH_E, CH)
    srcix_e = src.reshape(NC * NS, NCH_E // SCK, SCK, CH)
    dstix_e = dst.reshape(NC * NS, NCH_E // SCK, SCK, CH)
    srcix_f = src.reshape(NS, NCH_F // SCK, SCK, CH)
    dstix_f = dst.reshape(NS, NCH_F // SCK, SCK, CH)
    z128 = jnp.zeros((NP, D), jnp.float32)
    z16 = jnp.zeros((NP, DEGW), jnp.float32)
    b0r = b0.reshape(1, 256)
    b1r = b1.reshape(1, 256)

    degcat = _deg_call()(dstix_d, z16)
    y0, dinv = _scale_call()(x, degcat, degcat)

    acc1 = _prop_call(True)(srcix_e, dstix_e, y0, z128)
    y1 = _layer1_call()(acc1, acc1, y0, dinv, W0, b0r).reshape(2 * NP, D)

    acc2 = _prop_call(False)(srcix_f, dstix_f, y1, z128)
    y2 = _layer2_call()(acc2, acc2, y1, y1, dinv, W1, b1r).reshape(2 * NP, D)

    acc3 = _prop_call(False)(srcix_f, dstix_f, y2, z128)
    out = _final_call()(acc3, acc3, y2, y2, dinv, W1, b1r)
    return out


# speedup vs baseline: 1.1084x; 1.1084x over previous
import functools

import jax
import jax.numpy as jnp
from jax import lax
from jax.experimental import pallas as pl
from jax.experimental.pallas import tpu as pltpu
from jax.experimental.pallas import tpu_sc as plsc

N = 10000
NP = 10240
E = 320000
NC = 2
NS = 16
CH = 80
NCH_E = E // (NC * NS * CH)
NCH_F = E // (NS * CH)
RPT = NP // NS
SCK = 25
D = 128
DEGW = 16
BM = 1024
NBLK = NP // BM


def _sc_mesh():
    return plsc.VectorSubcoreMesh(core_axis_name="c", subcore_axis_name="s")


@functools.cache
def _deg_call():
    @functools.partial(
        pl.kernel,
        out_type=jax.ShapeDtypeStruct((NC, NP, DEGW), jnp.float32),
        mesh=_sc_mesh(),
        scratch_types=[
            pltpu.VMEM((NCH_E, CH), jnp.int32),
            pltpu.VMEM((CH, DEGW), jnp.float32),
            pltpu.VMEM_SHARED((NP, DEGW), jnp.float32),
            pltpu.SemaphoreType.DMA,
        ],
    )
    def deg_kernel(dstix, zeros, deg_out, didx_v, ones_v, acc, ssem):
        c = lax.axis_index("c")
        s = lax.axis_index("s")
        w = c * NS + s
        r0 = s * RPT

        @pl.loop(0, CH)
        def _(i):
            ones_v[i, :] = jnp.full((DEGW,), 1.0, jnp.float32)

        pltpu.sync_copy(zeros.at[pl.ds(r0, RPT)], acc.at[pl.ds(r0, RPT)])
        pltpu.sync_copy(dstix.at[w], didx_v)
        plsc.subcore_barrier()

        @pl.loop(0, NCH_E)
        def _(j):
            pltpu.async_copy(ones_v, acc.at[didx_v.at[j]], ssem, add=True)

        @pl.loop(0, NCH_E)
        def _(j):
            pltpu.make_async_copy(ones_v, acc.at[didx_v.at[0]], ssem).wait()

        plsc.subcore_barrier()
        pltpu.sync_copy(acc.at[pl.ds(r0, RPT)], deg_out.at[c, pl.ds(r0, RPT)])

    return deg_kernel


@functools.cache
def _prop_call(split_edges):
    nch = NCH_E if split_edges else NCH_F
    outer = nch // SCK

    @functools.partial(
        pl.kernel,
        out_type=jax.ShapeDtypeStruct((NC * NP, D), jnp.float32),
        mesh=_sc_mesh(),
        scratch_types=[
            pltpu.VMEM((SCK, CH), jnp.int32),
            pltpu.VMEM((SCK, CH), jnp.int32),
            pltpu.VMEM((3, CH, D), jnp.float32),
            pltpu.VMEM_SHARED((NP, D), jnp.float32),
            pltpu.SemaphoreType.DMA((3,)),
            pltpu.SemaphoreType.DMA((3,)),
        ],
    )
    def prop_kernel(srcix, dstix, table, zeros, out,
                    sidx_v, didx_v, msgs, acc, gsem, ssem):
        c = lax.axis_index("c")
        s = lax.axis_index("s")
        r0 = s * RPT
        cN = c * NP

        pltpu.sync_copy(zeros.at[pl.ds(r0, RPT)], acc.at[pl.ds(r0, RPT)])
        plsc.subcore_barrier()

        @pl.loop(0, outer)
        def _(o):
            if split_edges:
                w = c * NS + s
                pltpu.sync_copy(srcix.at[w, o], sidx_v)
                pltpu.sync_copy(dstix.at[w, o], didx_v)
            else:
                pltpu.sync_copy(srcix.at[s, o], sidx_v)
                pltpu.sync_copy(dstix.at[s, o], didx_v)

                @pl.loop(0, SCK)
                def _(i):
                    @pl.loop(0, CH // 16)
                    def _(k):
                        sidx_v[i, pl.ds(k * 16, 16)] = (
                            sidx_v[i, pl.ds(k * 16, 16)] + cN)

            @pl.loop(0, SCK + 1)
            def _(j):
                b = lax.rem(j, 3)

                @pl.when(j < SCK)
                def _():
                    @pl.when(j >= 3)
                    def _():
                        pltpu.make_async_copy(
                            msgs.at[b], acc.at[didx_v.at[0]],
                            ssem.at[b]).wait()

                    pltpu.async_copy(table.at[sidx_v.at[j]],
                                     msgs.at[b], gsem.at[b])

                @pl.when(j >= 1)
                def _():
                    pb = lax.rem(j + 2, 3)
                    pltpu.make_async_copy(table.at[sidx_v.at[0]],
                                          msgs.at[pb], gsem.at[pb]).wait()
                    pltpu.async_copy(msgs.at[pb], acc.at[didx_v.at[j - 1]],
                                     ssem.at[pb], add=True)

            for bb in range(3):
                pltpu.make_async_copy(msgs.at[bb], acc.at[didx_v.at[0]],
                                      ssem.at[bb]).wait()

        plsc.subcore_barrier()
        pltpu.sync_copy(acc.at[pl.ds(r0, RPT)], out.at[pl.ds(cN + r0, RPT)])

    return prop_kernel


def _scale_body(x_ref, da_ref, db_ref, y_ref, dinv_ref):
    deg = da_ref[0, :, :1] + db_ref[0, :, :1] + 1.0
    dinv = lax.rsqrt(deg)
    y_ref[...] = x_ref[...] * dinv
    dinv_ref[...] = jnp.broadcast_to(dinv, (BM, D))


@functools.cache
def _scale_call():
    return pl.pallas_call(
        _scale_body,
        grid=(NBLK,),
        in_specs=[
            pl.BlockSpec((BM, D), lambda i: (i, 0)),
            pl.BlockSpec((1, BM, DEGW), lambda i: (0, i, 0)),
            pl.BlockSpec((1, BM, DEGW), lambda i: (1, i, 0)),
        ],
        out_specs=[
            pl.BlockSpec((BM, D), lambda i: (i, 0)),
            pl.BlockSpec((BM, D), lambda i: (i, 0)),
        ],
        out_shape=[
            jax.ShapeDtypeStruct((NP, D), jnp.float32),
            jax.ShapeDtypeStruct((NP, D), jnp.float32),
        ],
    )


def _layer1_body(a0_ref, a1_ref, y_ref, dinv_ref, w_ref, b_ref, ycat_ref):
    dinv = dinv_ref[...]
    z = (a0_ref[...] + a1_ref[...] + y_ref[...]) * dinv
    h = jnp.dot(z, w_ref[...], preferred_element_type=jnp.float32) + b_ref[...]
    h = jnp.maximum(h, 0.0)
    ycat_ref[0] = h[:, :D] * dinv
    ycat_ref[1] = h[:, D:] * dinv


@functools.cache
def _layer1_call():
    return pl.pallas_call(
        _layer1_body,
        grid=(NBLK,),
        in_specs=[
            pl.BlockSpec((BM, D), lambda i: (i, 0)),
            pl.BlockSpec((BM, D), lambda i: (i + NBLK, 0)),
            pl.BlockSpec((BM, D), lambda i: (i, 0)),
            pl.BlockSpec((BM, D), lambda i: (i, 0)),
            pl.BlockSpec((D, 256), lambda i: (0, 0)),
            pl.BlockSpec((1, 256), lambda i: (0, 0)),
        ],
        out_specs=pl.BlockSpec((2, BM, D), lambda i: (0, i, 0)),
        out_shape=jax.ShapeDtypeStruct((2, NP, D), jnp.float32),
    )


def _layer2_body(a_lo, a_hi, y_lo, y_hi, dinv_ref, w_ref, b_ref, ycat_ref):
    dinv = dinv_ref[...]
    z = jnp.concatenate(
        [(a_lo[...] + y_lo[...]) * dinv, (a_hi[...] + y_hi[...]) * dinv],
        axis=1)
    h = jnp.dot(z, w_ref[...], preferred_element_type=jnp.float32) + b_ref[...]
    h = jnp.maximum(h, 0.0)
    ycat_ref[0] = h[:, :D] * dinv
    ycat_ref[1] = h[:, D:] * dinv


def _final_body(a_lo, a_hi, y_lo, y_hi, dinv_ref, w_ref, b_ref, out_ref):
    dinv = dinv_ref[...]
    z = jnp.concatenate(
        [(a_lo[...] + y_lo[...]) * dinv, (a_hi[...] + y_hi[...]) * dinv],
        axis=1)
    out_ref[...] = (
        jnp.dot(z, w_ref[...], preferred_element_type=jnp.float32) + b_ref[...])


def _wide_in_specs():
    return [
        pl.BlockSpec((BM, D), lambda i: (i, 0)),
        pl.BlockSpec((BM, D), lambda i: (i + NBLK, 0)),
        pl.BlockSpec((BM, D), lambda i: (i, 0)),
        pl.BlockSpec((BM, D), lambda i: (i + NBLK, 0)),
        pl.BlockSpec((BM, D), lambda i: (i, 0)),
        pl.BlockSpec((256, 256), lambda i: (0, 0)),
        pl.BlockSpec((1, 256), lambda i: (0, 0)),
    ]


@functools.cache
def _layer2_call():
    return pl.pallas_call(
        _layer2_body,
        grid=(NBLK,),
        in_specs=_wide_in_specs(),
        out_specs=pl.BlockSpec((2, BM, D), lambda i: (0, i, 0)),
        out_shape=jax.ShapeDtypeStruct((2, NP, D), jnp.float32),
    )


@functools.cache
def _final_call():
    return pl.pallas_call(
        _final_body,
        grid=(NBLK,),
        in_specs=_wide_in_specs(),
        out_specs=pl.BlockSpec((BM, 256), lambda i: (i, 0)),
        out_shape=jax.ShapeDtypeStruct((N, 256), jnp.float32),
    )


def kernel(x, edge_index, W0, b0, W1, b1):
    src = edge_index[0].astype(jnp.int32)
    dst = edge_index[1].astype(jnp.int32)
    dstix_d = dst.reshape(NC * NS, NCH_E, CH)
    srcix_e = src.reshape(NC * NS, NCH_E // SCK, SCK, CH)
    dstix_e = dst.reshape(NC * NS, NCH_E // SCK, SCK, CH)
    srcix_f = src.reshape(NS, NCH_F // SCK, SCK, CH)
    dstix_f = dst.reshape(NS, NCH_F // SCK, SCK, CH)
    z128 = jnp.zeros((NP, D), jnp.float32)
    z16 = jnp.zeros((NP, DEGW), jnp.float32)
    b0r = b0.reshape(1, 256)
    b1r = b1.reshape(1, 256)

    degcat = _deg_call()(dstix_d, z16)
    y0, dinv = _scale_call()(x, degcat, degcat)

    acc1 = _prop_call(True)(srcix_e, dstix_e, y0, z128)
    y1 = _layer1_call()(acc1, acc1, y0, dinv, W0, b0r).reshape(2 * NP, D)

    acc2 = _prop_call(False)(srcix_f, dstix_f, y1, z128)
    y2 = _layer2_call()(acc2, acc2, y1, y1, dinv, W1, b1r).reshape(2 * NP, D)

    acc3 = _prop_call(False)(srcix_f, dstix_f, y2, z128)
    out = _final_call()(acc3, acc3, y2, y2, dinv, W1, b1r)
    return out
